# baseline (speedup 1.0000x reference)
import functools

import jax
import jax.numpy as jnp
from jax import lax
from jax.experimental import pallas as pl
from jax.experimental.pallas import tpu as pltpu
from jax.experimental.pallas import tpu_sc as plsc

NC = 2
NS = 16
LANES = 8
NR = 4
NE = 6
C = 80


def _sc_spmm(n_nodes, n_edges, d, x, edata, wdata):
    n_workers = NC * NS
    epw = n_edges // n_workers
    nchunk = epw // C
    n_pad = 10240
    rows_per_tile = n_pad // NS
    nzero = rows_per_tile // C

    mesh = plsc.VectorSubcoreMesh(core_axis_name="c", subcore_axis_name="s")

    @functools.partial(
        pl.kernel,
        out_type=jax.ShapeDtypeStruct((NC, n_pad, d), jnp.float32),
        mesh=mesh,
        scratch_types=[
            pltpu.VMEM((NE, 2, C), jnp.int32),
            pltpu.VMEM((NE, C), jnp.float32),
            pltpu.VMEM((NR, C, d), jnp.float32),
            pltpu.VMEM_SHARED((n_pad, d), jnp.float32),
            pltpu.SemaphoreType.DMA,
            pltpu.SemaphoreType.DMA,
            pltpu.SemaphoreType.DMA,
        ],
    )
    def spmm(x_ref, ed_ref, wd_ref, out_ref, ebuf, wbuf, rows, acc,
             esem, gsem, ssem):
        cid = lax.axis_index("c")
        sid = lax.axis_index("s")
        wid = cid * NS + sid

        zero16 = jnp.zeros((16,), jnp.float32)

        def zrow(i, carry):
            for j in range(LANES):
                rows[0, i, pl.ds(j * 16, 16)] = zero16
            return carry

        lax.fori_loop(0, C, zrow, 0)
        row0 = sid * rows_per_tile
        for k in range(nzero):
            pltpu.sync_copy(rows.at[0], acc.at[pl.ds(row0 + k * C, C)])
        plsc.subcore_barrier()

        def issue_edma(g):
            pltpu.async_copy(ed_ref.at[wid, g], ebuf.at[g % NE], esem)
            pltpu.async_copy(wd_ref.at[wid, g], wbuf.at[g % NE], esem)

        def wait_edma(g):
            pltpu.make_async_copy(
                ed_ref.at[wid, g], ebuf.at[g % NE], esem).wait()
            pltpu.make_async_copy(
                wd_ref.at[wid, g], wbuf.at[g % NE], esem).wait()

        def issue_gather(g):
            pltpu.async_copy(
                x_ref.at[ebuf.at[g % NE, 0]], rows.at[g % NR], gsem)

        def wait_gather(g):
            pltpu.make_async_copy(
                x_ref.at[ebuf.at[g % NE, 0]], rows.at[g % NR], gsem).wait()

        def issue_scatter(g):
            pltpu.async_copy(
                rows.at[g % NR], acc.at[ebuf.at[g % NE, 1]], ssem, add=True)

        def wait_scatter(g):
            pltpu.make_async_copy(
                rows.at[g % NR], acc.at[ebuf.at[g % NE, 1]], ssem).wait()

        for g0 in range(3):
            issue_edma(g0)
        for g0 in range(2):
            wait_edma(g0)
            issue_gather(g0)

        def step(g, carry):
            b = g % NR
            wait_gather(g)

            for e16 in range(C // 16):
                w16 = wbuf[g % NE, pl.ds(e16 * 16, 16)]
                for i in range(16):
                    e = e16 * 16 + i
                    wv = w16.at[jnp.full((16,), i, jnp.int32)].get(
                        mode="promise_in_bounds")
                    for j in range(LANES):
                        sl = pl.ds(j * 16, 16)
                        rows[b, e, sl] = rows[b, e, sl] * wv
            issue_scatter(g)

            @pl.when(g + 2 < nchunk)
            def _():
                @pl.when(g >= 2)
                def _():
                    wait_scatter(g - 2)

                wait_edma(g + 2)
                issue_gather(g + 2)

            @pl.when(g + 3 < nchunk)
            def _():
                issue_edma(g + 3)

            return carry

        lax.fori_loop(0, nchunk, step, 0)
        for g0 in range(nchunk - 4, nchunk):
            wait_scatter(g0)

        plsc.subcore_barrier()
        pltpu.sync_copy(acc.at[pl.ds(row0, rows_per_tile)],
                        out_ref.at[cid, pl.ds(row0, rows_per_tile)])

    return spmm(x, edata, wdata)


def _tc_linear(partials, Wt, b2, n_nodes, d_out):
    n_pad = partials.shape[1]
    grid = 10
    rb = n_pad // grid

    def body(p_ref, wt_ref, b_ref, o_ref):
        acc = p_ref[0] + p_ref[1]
        o_ref[...] = jnp.maximum(
            jnp.dot(acc, wt_ref[...], preferred_element_type=jnp.float32)
            + b_ref[...], 0.0)

    return pl.pallas_call(
        body,
        grid=(grid,),
        in_specs=[
            pl.BlockSpec((2, rb, partials.shape[2]), lambda i: (0, i, 0)),
            pl.BlockSpec(Wt.shape, lambda i: (0, 0)),
            pl.BlockSpec((1, d_out), lambda i: (0, 0)),
        ],
        out_specs=pl.BlockSpec((rb, d_out), lambda i: (i, 0)),
        out_shape=jax.ShapeDtypeStruct((n_pad, d_out), jnp.float32),
    )(partials, Wt, b2)[:n_nodes]


def kernel(x, edge_index, edge_weight, W, b):
    n_nodes, d_in = x.shape
    n_edges = edge_weight.shape[0]
    d_out = W.shape[0]
    n_workers = NC * NS
    epw = n_edges // n_workers
    nchunk = epw // C
    ei = edge_index.astype(jnp.int32)
    edata = jnp.stack(
        [ei[0].reshape(n_workers, nchunk, C),
         ei[1].reshape(n_workers, nchunk, C)], axis=2)
    wdata = edge_weight.reshape(n_workers, nchunk, C)
    partials = _sc_spmm(n_nodes, n_edges, d_in, x, edata, wdata)
    return _tc_linear(partials, W.T, b.reshape(1, d_out), n_nodes, d_out)


# speedup vs baseline: 1.0772x; 1.0772x over previous
import functools

import jax
import jax.numpy as jnp
from jax import lax
from jax.experimental import pallas as pl
from jax.experimental.pallas import tpu as pltpu
from jax.experimental.pallas import tpu_sc as plsc

NC = 2
NS = 16
LANES = 8
NR = 4
NE = 6
C = 80


def _sc_spmm(n_nodes, n_edges, d, x, sdat, ddat, wdat):
    n_workers = NC * NS
    epw = n_edges // n_workers
    nchunk = epw // C
    n_pad = 10240
    rows_per_tile = n_pad // NS
    nzero = rows_per_tile // C

    mesh = plsc.VectorSubcoreMesh(core_axis_name="c", subcore_axis_name="s")

    @functools.partial(
        pl.kernel,
        out_type=jax.ShapeDtypeStruct((NC, n_pad, d), jnp.float32),
        mesh=mesh,
        scratch_types=[
            pltpu.VMEM((NE, C), jnp.int32),
            pltpu.VMEM((NE, C), jnp.int32),
            pltpu.VMEM((NE, C), jnp.float32),
            pltpu.VMEM((NR, C, d), jnp.float32),
            pltpu.VMEM_SHARED((n_pad, d), jnp.float32),
            pltpu.SemaphoreType.DMA,
            pltpu.SemaphoreType.DMA,
            pltpu.SemaphoreType.DMA,
        ],
    )
    def spmm(x_ref, sd_ref, dd_ref, wd_ref, out_ref, sbuf, dbuf, wbuf, rows,
             acc, esem, gsem, ssem):
        cid = lax.axis_index("c")
        sid = lax.axis_index("s")
        wid = cid * NS + sid

        zero16 = jnp.zeros((16,), jnp.float32)

        def zrow(i, carry):
            for j in range(LANES):
                rows[0, i, pl.ds(j * 16, 16)] = zero16
            return carry

        lax.fori_loop(0, C, zrow, 0)
        row0 = sid * rows_per_tile
        for k in range(nzero):
            pltpu.sync_copy(rows.at[0], acc.at[pl.ds(row0 + k * C, C)])
        plsc.subcore_barrier()

        def issue_edma(g):
            pltpu.async_copy(sd_ref.at[wid, g], sbuf.at[g % NE], esem)
            pltpu.async_copy(dd_ref.at[wid, g], dbuf.at[g % NE], esem)
            pltpu.async_copy(wd_ref.at[wid, g], wbuf.at[g % NE], esem)

        def wait_edma(g):
            pltpu.make_async_copy(
                sd_ref.at[wid, g], sbuf.at[g % NE], esem).wait()
            pltpu.make_async_copy(
                dd_ref.at[wid, g], dbuf.at[g % NE], esem).wait()
            pltpu.make_async_copy(
                wd_ref.at[wid, g], wbuf.at[g % NE], esem).wait()

        def issue_gather(g):
            pltpu.async_copy(
                x_ref.at[sbuf.at[g % NE]], rows.at[g % NR], gsem)

        def wait_gather(g):
            pltpu.make_async_copy(
                x_ref.at[sbuf.at[g % NE]], rows.at[g % NR], gsem).wait()

        def issue_scatter(g):
            pltpu.async_copy(
                rows.at[g % NR], acc.at[dbuf.at[g % NE]], ssem, add=True)

        def wait_scatter(g):
            pltpu.make_async_copy(
                rows.at[g % NR], acc.at[dbuf.at[g % NE]], ssem).wait()

        for g0 in range(3):
            issue_edma(g0)
        for g0 in range(2):
            wait_edma(g0)
            issue_gather(g0)

        def step(g, carry):
            b = g % NR
            wait_gather(g)

            for e16 in range(C // 16):
                w16 = wbuf[g % NE, pl.ds(e16 * 16, 16)]
                for i in range(16):
                    e = e16 * 16 + i
                    wv = w16.at[jnp.full((16,), i, jnp.int32)].get(
                        mode="promise_in_bounds")
                    for j in range(LANES):
                        sl = pl.ds(j * 16, 16)
                        rows[b, e, sl] = rows[b, e, sl] * wv
            issue_scatter(g)

            @pl.when(g + 2 < nchunk)
            def _():
                @pl.when(g >= 2)
                def _():
                    wait_scatter(g - 2)

                wait_edma(g + 2)
                issue_gather(g + 2)

            @pl.when(g + 3 < nchunk)
            def _():
                issue_edma(g + 3)

            return carry

        lax.fori_loop(0, nchunk, step, 0)
        for g0 in range(nchunk - 4, nchunk):
            wait_scatter(g0)

        plsc.subcore_barrier()
        pltpu.sync_copy(acc.at[pl.ds(row0, rows_per_tile)],
                        out_ref.at[cid, pl.ds(row0, rows_per_tile)])

    return spmm(x, sdat, ddat, wdat)


def _tc_linear(partials, Wt, b2, n_nodes, d_out):
    n_pad = partials.shape[1]
    grid = 10
    rb = n_pad // grid

    def body(p_ref, wt_ref, b_ref, o_ref):
        acc = p_ref[0] + p_ref[1]
        o_ref[...] = jnp.maximum(
            jnp.dot(acc, wt_ref[...], preferred_element_type=jnp.float32)
            + b_ref[...], 0.0)

    return pl.pallas_call(
        body,
        grid=(grid,),
        in_specs=[
            pl.BlockSpec((2, rb, partials.shape[2]), lambda i: (0, i, 0)),
            pl.BlockSpec(Wt.shape, lambda i: (0, 0)),
            pl.BlockSpec((1, d_out), lambda i: (0, 0)),
        ],
        out_specs=pl.BlockSpec((rb, d_out), lambda i: (i, 0)),
        out_shape=jax.ShapeDtypeStruct((n_pad, d_out), jnp.float32),
    )(partials, Wt, b2)[:n_nodes]


def kernel(x, edge_index, edge_weight, W, b):
    n_nodes, d_in = x.shape
    n_edges = edge_weight.shape[0]
    d_out = W.shape[0]
    n_workers = NC * NS
    epw = n_edges // n_workers
    nchunk = epw // C
    ei = edge_index.astype(jnp.int32)
    sdat = ei[0].reshape(n_workers, nchunk, C)
    ddat = ei[1].reshape(n_workers, nchunk, C)
    wdat = edge_weight.reshape(n_workers, nchunk, C)
    partials = _sc_spmm(n_nodes, n_edges, d_in, x, sdat, ddat, wdat)
    return _tc_linear(partials, W.T, b.reshape(1, d_out), n_nodes, d_out)


# speedup vs baseline: 1.1046x; 1.0254x over previous
import functools

import jax
import jax.numpy as jnp
from jax import lax
from jax.experimental import pallas as pl
from jax.experimental.pallas import tpu as pltpu
from jax.experimental.pallas import tpu_sc as plsc

NC = 2
NS = 16
LANES = 8
NR = 4
NE = 6
C = 80


def _sc_spmm(n_nodes, n_edges, d, x, sdat, ddat, wdat):
    n_workers = NC * NS
    epw = n_edges // n_workers
    nchunk = epw // C
    n_pad = 10240
    rows_per_tile = n_pad // NS
    nzero = rows_per_tile // C

    mesh = plsc.VectorSubcoreMesh(core_axis_name="c", subcore_axis_name="s")

    @functools.partial(
        pl.kernel,
        out_type=jax.ShapeDtypeStruct((NC, n_pad, d), jnp.float32),
        mesh=mesh,
        scratch_types=[
            pltpu.VMEM((NE, C), jnp.int32),
            pltpu.VMEM((NE, C), jnp.int32),
            pltpu.VMEM((NE, C), jnp.float32),
            pltpu.VMEM((NR, C, d), jnp.float32),
            pltpu.VMEM_SHARED((n_pad, d), jnp.float32),
            pltpu.SemaphoreType.DMA,
            pltpu.SemaphoreType.DMA,
            pltpu.SemaphoreType.DMA,
        ],
    )
    def spmm(x_ref, sd_ref, dd_ref, wd_ref, out_ref, sbuf, dbuf, wbuf, rows,
             acc, esem, gsem, ssem):
        cid = lax.axis_index("c")
        sid = lax.axis_index("s")
        wid = cid * NS + sid

        zero16 = jnp.zeros((16,), jnp.float32)

        def zrow(i, carry):
            for j in range(LANES):
                rows[0, i, pl.ds(j * 16, 16)] = zero16
            return carry

        lax.fori_loop(0, C, zrow, 0)
        row0 = sid * rows_per_tile
        for k in range(nzero):
            pltpu.sync_copy(rows.at[0], acc.at[pl.ds(row0 + k * C, C)])
        plsc.subcore_barrier()

        def issue_edma(g):
            pltpu.async_copy(sd_ref.at[wid, g], sbuf.at[g % NE], esem)
            pltpu.async_copy(dd_ref.at[wid, g], dbuf.at[g % NE], esem)
            pltpu.async_copy(wd_ref.at[wid, g], wbuf.at[g % NE], esem)

        def wait_edma(g):
            pltpu.make_async_copy(
                sd_ref.at[wid, g], sbuf.at[g % NE], esem).wait()
            pltpu.make_async_copy(
                dd_ref.at[wid, g], dbuf.at[g % NE], esem).wait()
            pltpu.make_async_copy(
                wd_ref.at[wid, g], wbuf.at[g % NE], esem).wait()

        def issue_gather(g):
            pltpu.async_copy(
                x_ref.at[sbuf.at[g % NE]], rows.at[g % NR], gsem)

        def wait_gather(g):
            pltpu.make_async_copy(
                x_ref.at[sbuf.at[g % NE]], rows.at[g % NR], gsem).wait()

        def issue_scatter(g):
            pltpu.async_copy(
                rows.at[g % NR], acc.at[dbuf.at[g % NE]], ssem, add=True)

        def wait_scatter(g):
            pltpu.make_async_copy(
                rows.at[g % NR], acc.at[dbuf.at[g % NE]], ssem).wait()

        for g0 in range(3):
            issue_edma(g0)
        for g0 in range(2):
            wait_edma(g0)
            issue_gather(g0)

        def step(g, carry):
            b = g % NR
            wait_gather(g)

            for e16 in range(C // 16):
                w16 = wbuf[g % NE, pl.ds(e16 * 16, 16)]
                for i in range(16):
                    e = e16 * 16 + i
                    wv = w16.at[jnp.full((16,), i, jnp.int32)].get(
                        mode="promise_in_bounds")
                    for j in range(LANES):
                        sl = pl.ds(j * 16, 16)
                        rows[b, e, sl] = rows[b, e, sl] * wv
            issue_scatter(g)

            @pl.when(g + 2 < nchunk)
            def _():
                @pl.when(g >= 2)
                def _():
                    wait_scatter(g - 2)

                wait_edma(g + 2)
                issue_gather(g + 2)

            @pl.when(g + 3 < nchunk)
            def _():
                issue_edma(g + 3)

            return carry

        lax.fori_loop(0, nchunk, step, 0)
        for g0 in range(nchunk - 4, nchunk):
            wait_scatter(g0)

        plsc.subcore_barrier()
        pltpu.sync_copy(acc.at[pl.ds(row0, rows_per_tile)],
                        out_ref.at[cid, pl.ds(row0, rows_per_tile)])

    return spmm(x, sdat, ddat, wdat)


def _tc_linear(partials, Wt, b2, n_nodes, d_out):
    grid = 10
    rb = n_nodes // grid

    def body(p_ref, wt_ref, b_ref, o_ref):
        acc = p_ref[0] + p_ref[1]
        o_ref[...] = jnp.maximum(
            jnp.dot(acc, wt_ref[...], preferred_element_type=jnp.float32)
            + b_ref[...], 0.0)

    return pl.pallas_call(
        body,
        grid=(grid,),
        in_specs=[
            pl.BlockSpec((2, rb, partials.shape[2]), lambda i: (0, i, 0)),
            pl.BlockSpec(Wt.shape, lambda i: (0, 0)),
            pl.BlockSpec((1, d_out), lambda i: (0, 0)),
        ],
        out_specs=pl.BlockSpec((rb, d_out), lambda i: (i, 0)),
        out_shape=jax.ShapeDtypeStruct((n_nodes, d_out), jnp.float32),
    )(partials, Wt, b2)


def kernel(x, edge_index, edge_weight, W, b):
    n_nodes, d_in = x.shape
    n_edges = edge_weight.shape[0]
    d_out = W.shape[0]
    n_workers = NC * NS
    epw = n_edges // n_workers
    nchunk = epw // C
    ei = edge_index.astype(jnp.int32)
    sdat = ei[0].reshape(n_workers, nchunk, C)
    ddat = ei[1].reshape(n_workers, nchunk, C)
    wdat = edge_weight.reshape(n_workers, nchunk, C)
    partials = _sc_spmm(n_nodes, n_edges, d_in, x, sdat, ddat, wdat)
    return _tc_linear(partials, W.T, b.reshape(1, d_out), n_nodes, d_out)


# speedup vs baseline: 1.1811x; 1.0693x over previous
import functools

import jax
import jax.numpy as jnp
from jax import lax
from jax.experimental import pallas as pl
from jax.experimental.pallas import tpu as pltpu
from jax.experimental.pallas import tpu_sc as plsc

NC = 2
NS = 16
LANES = 8
NR = 4
NE = 6
C = 80


def _sc_spmm(n_nodes, n_edges, d, x, sddat, wdat):
    n_workers = NC * NS
    epw = n_edges // n_workers
    nchunk = epw // C
    n_pad = 10240
    rows_per_tile = n_pad // NS
    nzero = rows_per_tile // C

    mesh = plsc.VectorSubcoreMesh(core_axis_name="c", subcore_axis_name="s")

    @functools.partial(
        pl.kernel,
        out_type=jax.ShapeDtypeStruct((NC, n_pad, d), jnp.float32),
        mesh=mesh,
        scratch_types=[
            pltpu.VMEM((NE, C), jnp.int32),
            pltpu.VMEM((NE, C), jnp.int32),
            pltpu.VMEM((NE, C), jnp.float32),
            pltpu.VMEM((NR, C, d), jnp.float32),
            pltpu.VMEM_SHARED((n_pad, d), jnp.float32),
            pltpu.SemaphoreType.DMA,
            pltpu.SemaphoreType.DMA,
            pltpu.SemaphoreType.DMA,
        ],
    )
    def spmm(x_ref, sd_ref, wd_ref, out_ref, sbuf, dbuf, wbuf, rows,
             acc, esem, gsem, ssem):
        cid = lax.axis_index("c")
        sid = lax.axis_index("s")
        wid = cid * NS + sid

        def issue_edma(g):
            pltpu.async_copy(sd_ref.at[wid, g], sbuf.at[g % NE], esem)
            pltpu.async_copy(sd_ref.at[n_workers + wid, g], dbuf.at[g % NE],
                             esem)
            pltpu.async_copy(wd_ref.at[wid, g], wbuf.at[g % NE], esem)

        def wait_edma(g):
            pltpu.make_async_copy(
                sd_ref.at[wid, g], sbuf.at[g % NE], esem).wait()
            pltpu.make_async_copy(
                sd_ref.at[n_workers + wid, g], dbuf.at[g % NE], esem).wait()
            pltpu.make_async_copy(
                wd_ref.at[wid, g], wbuf.at[g % NE], esem).wait()

        def issue_gather(g):
            pltpu.async_copy(
                x_ref.at[sbuf.at[g % NE]], rows.at[g % NR], gsem)

        def wait_gather(g):
            pltpu.make_async_copy(
                x_ref.at[sbuf.at[g % NE]], rows.at[g % NR], gsem).wait()

        def issue_scatter(g):
            pltpu.async_copy(
                rows.at[g % NR], acc.at[dbuf.at[g % NE]], ssem, add=True)

        def wait_scatter(g):
            pltpu.make_async_copy(
                rows.at[g % NR], acc.at[dbuf.at[g % NE]], ssem).wait()

        for g0 in range(3):
            issue_edma(g0)
        zero16 = jnp.zeros((16,), jnp.float32)

        def zrow(i, carry):
            for j in range(LANES):
                rows[0, i, pl.ds(j * 16, 16)] = zero16
            return carry

        lax.fori_loop(0, C, zrow, 0)
        row0 = sid * rows_per_tile
        for k in range(nzero):
            pltpu.sync_copy(rows.at[0], acc.at[pl.ds(row0 + k * C, C)])
        plsc.subcore_barrier()
        for g0 in range(2):
            wait_edma(g0)
            issue_gather(g0)

        def step(g, carry):
            b = g % NR
            wait_gather(g)

            for e16 in range(C // 16):
                w16 = wbuf[g % NE, pl.ds(e16 * 16, 16)]
                for i in range(16):
                    e = e16 * 16 + i
                    wv = w16.at[jnp.full((16,), i, jnp.int32)].get(
                        mode="promise_in_bounds")
                    for j in range(LANES):
                        sl = pl.ds(j * 16, 16)
                        rows[b, e, sl] = rows[b, e, sl] * wv
            issue_scatter(g)

            @pl.when(g + 2 < nchunk)
            def _():
                @pl.when(g >= 2)
                def _():
                    wait_scatter(g - 2)

                wait_edma(g + 2)
                issue_gather(g + 2)

            @pl.when(g + 3 < nchunk)
            def _():
                issue_edma(g + 3)

            return carry

        lax.fori_loop(0, nchunk, step, 0)
        for g0 in range(nchunk - 4, nchunk):
            wait_scatter(g0)

        plsc.subcore_barrier()
        pltpu.sync_copy(acc.at[pl.ds(row0, rows_per_tile)],
                        out_ref.at[cid, pl.ds(row0, rows_per_tile)])

    return spmm(x, sddat, wdat)


def _tc_linear(partials, Wt, b2, n_nodes, d_out):
    grid = 10
    rb = n_nodes // grid

    def body(p_ref, wt_ref, b_ref, o_ref):
        acc = p_ref[0] + p_ref[1]
        o_ref[...] = jnp.maximum(
            jnp.dot(acc, wt_ref[...], preferred_element_type=jnp.float32)
            + b_ref[...], 0.0)

    return pl.pallas_call(
        body,
        grid=(grid,),
        in_specs=[
            pl.BlockSpec((2, rb, partials.shape[2]), lambda i: (0, i, 0)),
            pl.BlockSpec(Wt.shape, lambda i: (0, 0)),
            pl.BlockSpec((1, d_out), lambda i: (0, 0)),
        ],
        out_specs=pl.BlockSpec((rb, d_out), lambda i: (i, 0)),
        out_shape=jax.ShapeDtypeStruct((n_nodes, d_out), jnp.float32),
    )(partials, Wt, b2)


def kernel(x, edge_index, edge_weight, W, b):
    n_nodes, d_in = x.shape
    n_edges = edge_weight.shape[0]
    d_out = W.shape[0]
    n_workers = NC * NS
    epw = n_edges // n_workers
    nchunk = epw // C
    ei = edge_index.astype(jnp.int32)
    sddat = ei.reshape(2 * n_workers, nchunk, C)
    wdat = edge_weight.reshape(n_workers, nchunk, C)
    partials = _sc_spmm(n_nodes, n_edges, d_in, x, sddat, wdat)
    return _tc_linear(partials, W.T, b.reshape(1, d_out), n_nodes, d_out)


# speedup vs baseline: 1.2530x; 1.0609x over previous
import functools

import jax
import jax.numpy as jnp
from jax import lax
from jax.experimental import pallas as pl
from jax.experimental.pallas import tpu as pltpu
from jax.experimental.pallas import tpu_sc as plsc

NC = 2
NS = 16
LANES = 8
NR = 4
NE = 6
C = 80


def _sc_spmm(n_nodes, n_edges, d, x, sddat, wdat):
    n_workers = NC * NS
    epw = n_edges // n_workers
    nchunk = epw // C
    n_pad = 10240
    rows_per_tile = n_pad // NS
    nzero = rows_per_tile // C

    mesh = plsc.VectorSubcoreMesh(core_axis_name="c", subcore_axis_name="s")

    @functools.partial(
        pl.kernel,
        out_type=jax.ShapeDtypeStruct((NC, n_pad, d), jnp.float32),
        mesh=mesh,
        scratch_types=[
            pltpu.VMEM((NE, C), jnp.int32),
            pltpu.VMEM((NE, C), jnp.int32),
            pltpu.VMEM((NE, C), jnp.float32),
            pltpu.VMEM((NR, C, d), jnp.float32),
            pltpu.VMEM_SHARED((n_pad, d), jnp.float32),
            pltpu.SemaphoreType.DMA,
            pltpu.SemaphoreType.DMA,
            pltpu.SemaphoreType.DMA,
        ],
    )
    def spmm(x_ref, sd_ref, wd_ref, out_ref, sbuf, dbuf, wbuf, rows,
             acc, esem, gsem, ssem):
        cid = lax.axis_index("c")
        sid = lax.axis_index("s")
        wid = cid * NS + sid

        def issue_edma(g):
            pltpu.async_copy(sd_ref.at[wid, g], sbuf.at[g % NE], esem)
            pltpu.async_copy(sd_ref.at[n_workers + wid, g], dbuf.at[g % NE],
                             esem)
            pltpu.async_copy(wd_ref.at[wid, g], wbuf.at[g % NE], esem)

        def wait_edma(g):
            pltpu.make_async_copy(
                sd_ref.at[wid, g], sbuf.at[g % NE], esem).wait()
            pltpu.make_async_copy(
                sd_ref.at[n_workers + wid, g], dbuf.at[g % NE], esem).wait()
            pltpu.make_async_copy(
                wd_ref.at[wid, g], wbuf.at[g % NE], esem).wait()

        def issue_gather(g):
            pltpu.async_copy(
                x_ref.at[sbuf.at[g % NE]], rows.at[g % NR], gsem)

        def wait_gather(g):
            pltpu.make_async_copy(
                x_ref.at[sbuf.at[g % NE]], rows.at[g % NR], gsem).wait()

        def issue_scatter(g):
            pltpu.async_copy(
                rows.at[g % NR], acc.at[dbuf.at[g % NE]], ssem, add=True)

        def wait_scatter(g):
            pltpu.make_async_copy(
                rows.at[g % NR], acc.at[dbuf.at[g % NE]], ssem).wait()

        for g0 in range(3):
            issue_edma(g0)
        zero16 = jnp.zeros((16,), jnp.float32)

        def zrow(i, carry):
            for j in range(LANES):
                rows[0, i, pl.ds(j * 16, 16)] = zero16
            return carry

        lax.fori_loop(0, C, zrow, 0)
        row0 = sid * rows_per_tile
        for k in range(nzero):
            pltpu.sync_copy(rows.at[0], acc.at[pl.ds(row0 + k * C, C)])
        plsc.subcore_barrier()
        for g0 in range(2):
            wait_edma(g0)
            issue_gather(g0)

        def step(g, carry):
            b = g % NR
            wait_gather(g)

            @pl.when(g >= 2)
            def _():
                wait_scatter(g - 2)

            @pl.when(g + 2 < nchunk)
            def _():
                wait_edma(g + 2)
                issue_gather(g + 2)

            @pl.when(g + 3 < nchunk)
            def _():
                issue_edma(g + 3)

            for e16 in range(C // 16):
                w16 = wbuf[g % NE, pl.ds(e16 * 16, 16)]
                for i in range(16):
                    e = e16 * 16 + i
                    wv = w16.at[jnp.full((16,), i, jnp.int32)].get(
                        mode="promise_in_bounds")
                    for j in range(LANES):
                        sl = pl.ds(j * 16, 16)
                        rows[b, e, sl] = rows[b, e, sl] * wv
            issue_scatter(g)
            return carry

        lax.fori_loop(0, nchunk, step, 0)
        for g0 in range(nchunk - 2, nchunk):
            wait_scatter(g0)

        plsc.subcore_barrier()
        pltpu.sync_copy(acc.at[pl.ds(row0, rows_per_tile)],
                        out_ref.at[cid, pl.ds(row0, rows_per_tile)])

    return spmm(x, sddat, wdat)


def _tc_linear(partials, Wt, b2, n_nodes, d_out):
    grid = 10
    rb = n_nodes // grid

    def body(p_ref, wt_ref, b_ref, o_ref):
        acc = p_ref[0] + p_ref[1]
        o_ref[...] = jnp.maximum(
            jnp.dot(acc, wt_ref[...], preferred_element_type=jnp.float32)
            + b_ref[...], 0.0)

    return pl.pallas_call(
        body,
        grid=(grid,),
        in_specs=[
            pl.BlockSpec((2, rb, partials.shape[2]), lambda i: (0, i, 0)),
            pl.BlockSpec(Wt.shape, lambda i: (0, 0)),
            pl.BlockSpec((1, d_out), lambda i: (0, 0)),
        ],
        out_specs=pl.BlockSpec((rb, d_out), lambda i: (i, 0)),
        out_shape=jax.ShapeDtypeStruct((n_nodes, d_out), jnp.float32),
    )(partials, Wt, b2)


def kernel(x, edge_index, edge_weight, W, b):
    n_nodes, d_in = x.shape
    n_edges = edge_weight.shape[0]
    d_out = W.shape[0]
    n_workers = NC * NS
    epw = n_edges // n_workers
    nchunk = epw // C
    ei = edge_index.astype(jnp.int32)
    sddat = ei.reshape(2 * n_workers, nchunk, C)
    wdat = edge_weight.reshape(n_workers, nchunk, C)
    partials = _sc_spmm(n_nodes, n_edges, d_in, x, sddat, wdat)
    return _tc_linear(partials, W.T, b.reshape(1, d_out), n_nodes, d_out)


# speedup vs baseline: 1.2587x; 1.0046x over previous
import functools

import jax
import jax.numpy as jnp
from jax import lax
from jax.experimental import pallas as pl
from jax.experimental.pallas import tpu as pltpu
from jax.experimental.pallas import tpu_sc as plsc

NC = 2
NS = 16
LANES = 8
NR = 4
NE = 6
C = 80


def _sc_spmm(n_nodes, n_edges, d, x, sddat, wdat):
    n_workers = NC * NS
    epw = n_edges // n_workers
    nchunk = epw // C
    n_pad = 10240
    rows_per_tile = n_pad // NS
    nzero = rows_per_tile // C

    mesh = plsc.VectorSubcoreMesh(core_axis_name="c", subcore_axis_name="s")

    @functools.partial(
        pl.kernel,
        out_type=jax.ShapeDtypeStruct((NC, n_pad, d), jnp.float32),
        mesh=mesh,
        scratch_types=[
            pltpu.VMEM((NE, C), jnp.int32),
            pltpu.VMEM((NE, C), jnp.int32),
            pltpu.VMEM((NE, C), jnp.float32),
            pltpu.VMEM((NR, C, d), jnp.float32),
            pltpu.VMEM_SHARED((n_pad, d), jnp.float32),
            pltpu.SemaphoreType.DMA,
            pltpu.SemaphoreType.DMA,
            pltpu.SemaphoreType.DMA,
        ],
    )
    def spmm(x_ref, sd_ref, wd_ref, out_ref, sbuf, dbuf, wbuf, rows,
             acc, esem, gsem, ssem):
        cid = lax.axis_index("c")
        sid = lax.axis_index("s")
        wid = cid * NS + sid

        def issue_edma(g):
            pltpu.async_copy(sd_ref.at[wid, g], sbuf.at[g % NE], esem)
            pltpu.async_copy(sd_ref.at[n_workers + wid, g], dbuf.at[g % NE],
                             esem)
            pltpu.async_copy(wd_ref.at[wid, g], wbuf.at[g % NE], esem)

        def wait_edma(g):
            pltpu.make_async_copy(
                sd_ref.at[wid, g], sbuf.at[g % NE], esem).wait()
            pltpu.make_async_copy(
                sd_ref.at[n_workers + wid, g], dbuf.at[g % NE], esem).wait()
            pltpu.make_async_copy(
                wd_ref.at[wid, g], wbuf.at[g % NE], esem).wait()

        def issue_gather(g):
            pltpu.async_copy(
                x_ref.at[sbuf.at[g % NE]], rows.at[(g + 1) % NR], gsem)

        def wait_gather(g):
            pltpu.make_async_copy(
                x_ref.at[sbuf.at[g % NE]], rows.at[(g + 1) % NR], gsem).wait()

        def issue_scatter(g):
            pltpu.async_copy(
                rows.at[(g + 1) % NR], acc.at[dbuf.at[g % NE]], ssem,
                add=True)

        def wait_scatter(g):
            pltpu.make_async_copy(
                rows.at[(g + 1) % NR], acc.at[dbuf.at[g % NE]], ssem).wait()

        for g0 in range(3):
            issue_edma(g0)
        for g0 in range(2):
            wait_edma(g0)
            issue_gather(g0)
        zero16 = jnp.zeros((16,), jnp.float32)

        def zrow(i, carry):
            for j in range(LANES):
                rows[0, i, pl.ds(j * 16, 16)] = zero16
            return carry

        lax.fori_loop(0, C, zrow, 0)
        row0 = sid * rows_per_tile
        for k in range(nzero):
            pltpu.sync_copy(rows.at[0], acc.at[pl.ds(row0 + k * C, C)])
        plsc.subcore_barrier()

        def step(g, carry):
            b = (g + 1) % NR
            wait_gather(g)

            @pl.when(g >= 2)
            def _():
                wait_scatter(g - 2)

            @pl.when(g + 2 < nchunk)
            def _():
                wait_edma(g + 2)
                issue_gather(g + 2)

            @pl.when(g + 3 < nchunk)
            def _():
                issue_edma(g + 3)

            for e16 in range(C // 16):
                w16 = wbuf[g % NE, pl.ds(e16 * 16, 16)]
                for i in range(16):
                    e = e16 * 16 + i
                    wv = w16.at[jnp.full((16,), i, jnp.int32)].get(
                        mode="promise_in_bounds")
                    for j in range(LANES):
                        sl = pl.ds(j * 16, 16)
                        rows[b, e, sl] = rows[b, e, sl] * wv
            issue_scatter(g)
            return carry

        lax.fori_loop(0, nchunk, step, 0)
        for g0 in range(nchunk - 2, nchunk):
            wait_scatter(g0)

        plsc.subcore_barrier()
        pltpu.sync_copy(acc.at[pl.ds(row0, rows_per_tile)],
                        out_ref.at[cid, pl.ds(row0, rows_per_tile)])

    return spmm(x, sddat, wdat)


def _tc_linear(partials, Wt, b2, n_nodes, d_out):
    grid = 10
    rb = n_nodes // grid

    def body(p_ref, wt_ref, b_ref, o_ref):
        acc = p_ref[0] + p_ref[1]
        o_ref[...] = jnp.maximum(
            jnp.dot(acc, wt_ref[...], preferred_element_type=jnp.float32)
            + b_ref[...], 0.0)

    return pl.pallas_call(
        body,
        grid=(grid,),
        in_specs=[
            pl.BlockSpec((2, rb, partials.shape[2]), lambda i: (0, i, 0)),
            pl.BlockSpec(Wt.shape, lambda i: (0, 0)),
            pl.BlockSpec((1, d_out), lambda i: (0, 0)),
        ],
        out_specs=pl.BlockSpec((rb, d_out), lambda i: (i, 0)),
        out_shape=jax.ShapeDtypeStruct((n_nodes, d_out), jnp.float32),
    )(partials, Wt, b2)


def kernel(x, edge_index, edge_weight, W, b):
    n_nodes, d_in = x.shape
    n_edges = edge_weight.shape[0]
    d_out = W.shape[0]
    n_workers = NC * NS
    epw = n_edges // n_workers
    nchunk = epw // C
    ei = edge_index.astype(jnp.int32)
    sddat = ei.reshape(2 * n_workers, nchunk, C)
    wdat = edge_weight.reshape(n_workers, nchunk, C)
    partials = _sc_spmm(n_nodes, n_edges, d_in, x, sddat, wdat)
    return _tc_linear(partials, W.T, b.reshape(1, d_out), n_nodes, d_out)
